# Initial kernel scaffold; baseline (speedup 1.0000x reference)
import jax
import jax.numpy as jnp
from jax.experimental import pallas as pl


def kernel(pos, batch, params):
    raise NotImplementedError("write your pallas kernel here")


# speedup vs baseline: 1.2042x; 1.2042x over previous
import functools

import jax
import jax.numpy as jnp
from jax import lax
from jax.experimental import pallas as pl
from jax.experimental.pallas import tpu as pltpu
from jax.experimental.pallas import tpu_sc as plsc

_RATIO1, _RATIO2 = 0.5, 0.25
_R1, _R2 = 0.2, 0.4
_MAX_NB = 64
_N, _B = 8192, 8

_NEG = float("-inf")


def _r2_f32(r):
    import numpy as np
    t = float(r) * float(r)
    t32 = np.float32(t)
    if float(t32) > t:
        t32 = np.nextafter(t32, np.float32(0.0))
    return t32


def _fps_body(mmax_ref, px_ref, py_ref, pz_ref, vm_ref, out_ref, dist_ref):
    B, P = px_ref.shape
    px = px_ref[...]
    py = py_ref[...]
    pz = pz_ref[...]
    d0 = ((px - px[:, 0:1]) ** 2 + (py - py[:, 0:1]) ** 2
          + (pz - pz[:, 0:1]) ** 2)
    dist_ref[...] = jnp.where(vm_ref[...] > 0, d0, _NEG)
    jidx = lax.broadcasted_iota(jnp.int32, (B, P), 1)
    lane = lax.broadcasted_iota(jnp.int32, (B, 128), 1)
    mmax = mmax_ref[0]

    def it(i, buf):
        dist = dist_ref[...]
        maxv = jnp.max(dist, axis=1, keepdims=True)
        sel = dist >= maxv
        jsel = jnp.min(jnp.where(sel, jidx, P), axis=1, keepdims=True)
        onehot = jidx == jsel
        qx = jnp.max(jnp.where(onehot, px, _NEG), axis=1, keepdims=True)
        qy = jnp.max(jnp.where(onehot, py, _NEG), axis=1, keepdims=True)
        qz = jnp.max(jnp.where(onehot, pz, _NEG), axis=1, keepdims=True)
        d2 = (px - qx) ** 2 + (py - qy) ** 2 + (pz - qz) ** 2
        dist_ref[...] = jnp.minimum(dist, d2)
        im = lax.rem(i, 128)
        buf = jnp.where(lane == im, jsel, buf)

        @pl.when(im == 127)
        def _flush():
            base = pl.multiple_of(i - 127, 128)
            out_ref[:, pl.ds(base, 128)] = buf

        return buf

    buf0 = jnp.zeros((B, 128), jnp.int32)
    buf = lax.fori_loop(1, mmax, it, buf0)
    last_base = pl.multiple_of(((mmax - 1) // 128) * 128, 128)
    out_ref[:, pl.ds(last_base, 128)] = buf


def _fps_pallas(px, py, pz, vmask, mmax, mcap, interpret=False):
    B, P = px.shape
    return pl.pallas_call(
        _fps_body,
        out_shape=jax.ShapeDtypeStruct((B, mcap), jnp.int32),
        in_specs=[
            pl.BlockSpec(memory_space=pltpu.SMEM),
            pl.BlockSpec(memory_space=pltpu.VMEM),
            pl.BlockSpec(memory_space=pltpu.VMEM),
            pl.BlockSpec(memory_space=pltpu.VMEM),
            pl.BlockSpec(memory_space=pltpu.VMEM),
        ],
        out_specs=pl.BlockSpec(memory_space=pltpu.VMEM),
        scratch_shapes=[pltpu.VMEM((B, P), jnp.float32)],
        interpret=interpret,
    )(jnp.reshape(mmax.astype(jnp.int32), (1,)), px, py, pz, vmask)


def _fps_level(pos_all, starts, counts, ratio, cap, ppad, mcap, interpret=False):
    P0 = pos_all.shape[0]
    mvec = jnp.maximum(1, jnp.ceil(ratio * counts.astype(jnp.float32)).astype(counts.dtype))
    offs = jnp.concatenate([jnp.zeros((1,), mvec.dtype), jnp.cumsum(mvec)])
    total = offs[-1]

    j = jnp.arange(ppad)
    gidx = jnp.minimum(starts[:, None] + j[None, :], P0 - 1)
    px = pos_all[:, 0][gidx]
    py = pos_all[:, 1][gidx]
    pz = pos_all[:, 2][gidx]
    vmask = (j[None, :] < counts[:, None]).astype(jnp.float32)
    mmax = jnp.max(mvec)

    jsel = _fps_pallas(px, py, pz, vmask, mmax, mcap, interpret=interpret)

    t = jnp.arange(cap)
    b = jnp.clip(jnp.searchsorted(offs, t, side="right") - 1, 0, _B - 1)
    i = jnp.clip(t - offs[b], 0, mcap - 1)
    jsel_t = jsel[b, i]
    gsel = jnp.minimum(starts[b] + jsel_t, P0 - 1)
    valid = t < total
    idx = jnp.where(valid, gsel, 0)
    bid = jnp.where(valid, b, 0).astype(starts.dtype)
    return idx, bid, valid, mvec, offs, jsel


_NBW = 80


def _radius_sc_call(qx, qy, qz, qs, qc, tx, ty, tz, thr, mpad, tlen):
    info = plsc.get_sparse_core_info()
    NW = info.num_cores * info.num_subcores
    L = info.num_lanes
    qpw = mpad // NW
    mesh = plsc.VectorSubcoreMesh(core_axis_name="c", subcore_axis_name="s")

    @functools.partial(
        pl.kernel, mesh=mesh,
        out_type=jax.ShapeDtypeStruct((mpad * _NBW,), jnp.int32),
        scratch_types=[
            pltpu.VMEM((tlen,), jnp.float32),
            pltpu.VMEM((tlen,), jnp.float32),
            pltpu.VMEM((tlen,), jnp.float32),
            pltpu.VMEM((qpw,), jnp.float32),
            pltpu.VMEM((qpw,), jnp.float32),
            pltpu.VMEM((qpw,), jnp.float32),
            pltpu.VMEM((qpw,), jnp.int32),
            pltpu.VMEM((qpw,), jnp.int32),
            pltpu.VMEM((qpw * _NBW,), jnp.int32),
        ],
    )
    def k(qx_h, qy_h, qz_h, qs_h, qc_h, tx_h, ty_h, tz_h, out_h,
          txv, tyv, tzv, qxv, qyv, qzv, qsv, qcv, nbrv):
        wid = lax.axis_index("s") * info.num_cores + lax.axis_index("c")
        base = wid * qpw
        pltpu.sync_copy(tx_h, txv)
        pltpu.sync_copy(ty_h, tyv)
        pltpu.sync_copy(tz_h, tzv)
        pltpu.sync_copy(qx_h.at[pl.ds(base, qpw)], qxv)
        pltpu.sync_copy(qy_h.at[pl.ds(base, qpw)], qyv)
        pltpu.sync_copy(qz_h.at[pl.ds(base, qpw)], qzv)
        pltpu.sync_copy(qs_h.at[pl.ds(base, qpw)], qsv)
        pltpu.sync_copy(qc_h.at[pl.ds(base, qpw)], qcv)
        lanes = lax.broadcasted_iota(jnp.int32, (L,), 0)
        neg1 = jnp.full((L,), -1, jnp.int32)

        def per_query(q, carry):
            qxs = qxv[q]
            qys = qyv[q]
            qzs = qzv[q]
            s0 = qsv[q]
            cnt = qcv[q]
            rowbase = q * _NBW
            for s in range(4):
                nbrv[pl.ds(rowbase + s * L, L)] = neg1
            c0 = s0 // L
            c1 = (s0 + cnt + L - 1) // L

            def cond(st):
                c, off = st
                return (c < c1) & (off < _MAX_NB)

            def body(st):
                c, off = st
                absj = lanes + c * L
                x = txv[pl.ds(c * L, L)]
                y = tyv[pl.ds(c * L, L)]
                z = tzv[pl.ds(c * L, L)]
                d2 = (x - qxs) ** 2 + (y - qys) ** 2 + (z - qzs) ** 2
                m = (absj >= s0) & (absj < s0 + cnt) & (d2 <= thr)
                pcnt = jnp.sum(m.astype(jnp.int32))
                plsc.store_compressed(nbrv.at[pl.ds(rowbase + off, L)], absj, m)
                return c + 1, off + pcnt

            lax.while_loop(cond, body, (c0, jnp.int32(0)))
            return carry

        lax.fori_loop(0, qpw, per_query, 0)
        pltpu.sync_copy(nbrv, out_h.at[pl.ds(base * _NBW, qpw * _NBW)])

    flat = k(qx, qy, qz, qs, qc, tx, ty, tz)
    return flat.reshape(mpad, _NBW)


def _radius_sc(pos_all, starts, counts, q_idx, q_bid, q_val, thr, max_nb,
               mpad, tlen):
    M = q_idx.shape[0]
    P0 = pos_all.shape[0]
    qpos = pos_all[q_idx]
    qs = starts[q_bid].astype(jnp.int32)
    qc = jnp.where(q_val, counts[q_bid], 0).astype(jnp.int32)
    pad = mpad - M
    qx = jnp.pad(qpos[:, 0], (0, pad))
    qy = jnp.pad(qpos[:, 1], (0, pad))
    qz = jnp.pad(qpos[:, 2], (0, pad))
    qs = jnp.pad(qs, (0, pad))
    qc = jnp.pad(qc, (0, pad))
    tpad = tlen - P0
    tx = jnp.pad(pos_all[:, 0], (0, tpad))
    ty = jnp.pad(pos_all[:, 1], (0, tpad))
    tz = jnp.pad(pos_all[:, 2], (0, tpad))
    nbr_raw = _radius_sc_call(qx, qy, qz, qs, qc, tx, ty, tz,
                              float(thr), mpad, tlen)[:M, :max_nb]
    msk = nbr_raw >= 0
    nbr = jnp.where(msk, nbr_raw, 0)
    return nbr, msk


def _radius_xla(pos_all, starts, counts, q_idx, q_bid, q_val, thr, max_nb):
    P = pos_all.shape[0]
    j = jnp.arange(P)

    def one(qi, qb, qv):
        g = jnp.minimum(starts[qb] + j, P - 1)
        d2 = jnp.sum((pos_all[g] - pos_all[qi]) ** 2, axis=-1)
        cond = (j < counts[qb]) & (d2 <= thr) & qv
        rank = jnp.cumsum(cond.astype(jnp.int32)) - 1
        slot = jnp.where(cond & (rank < max_nb), rank, max_nb)
        nbr = jnp.zeros((max_nb + 1,), g.dtype).at[slot].set(g)[:max_nb]
        cnt = jnp.minimum(jnp.sum(cond.astype(jnp.int32)), max_nb)
        msk = jnp.arange(max_nb) < cnt
        return nbr, msk

    return jax.vmap(one)(q_idx, q_bid, q_val)


def _masked_bn(h, mask, g, be, eps=1e-5):
    m = mask[..., None].astype(h.dtype)
    cnt = jnp.maximum(jnp.sum(m), 1.0)
    mean = jnp.sum(h * m, axis=(0, 1)) / cnt
    var = jnp.sum(((h - mean) ** 2) * m, axis=(0, 1)) / cnt
    return (h - mean) / jnp.sqrt(var + eps) * g + be


def _mlp_edge(h, layers, mask):
    for i, lp in enumerate(layers):
        h = h @ lp["W"] + lp["b"]
        if i < len(layers) - 1:
            h = _masked_bn(h, mask, lp["g"], lp["be"])
            h = jax.nn.relu(h)
    return h


def _mlp_node(h, layers, use_bn):
    for i, lp in enumerate(layers):
        h = h @ lp["W"] + lp["b"]
        if i < len(layers) - 1:
            if use_bn:
                mean = jnp.mean(h, axis=0)
                var = jnp.mean((h - mean) ** 2, axis=0)
                h = (h - mean) / jnp.sqrt(var + 1e-5) * lp["g"] + lp["be"]
            h = jax.nn.relu(h)
    return h


def _mlp_node_masked(h, layers, valid, eps=1e-5):
    m = valid[:, None].astype(h.dtype)
    cnt = jnp.maximum(jnp.sum(m), 1.0)
    for i, lp in enumerate(layers):
        h = h @ lp["W"] + lp["b"]
        if i < len(layers) - 1:
            mean = jnp.sum(h * m, axis=0) / cnt
            var = jnp.sum(((h - mean) ** 2) * m, axis=0) / cnt
            h = (h - mean) / jnp.sqrt(var + eps) * lp["g"] + lp["be"]
            h = jax.nn.relu(h)
    return h


def _masked_max(h, mask):
    return jnp.max(jnp.where(mask[..., None], h, jnp.finfo(h.dtype).min), axis=1)


def kernel(pos, batch, params):
    n = pos.shape[0]
    counts = jnp.bincount(batch, length=_B)
    starts = jnp.concatenate([jnp.zeros((1,), counts.dtype), jnp.cumsum(counts)[:-1]])
    cap1 = n // 2 + _B
    ppad1, mcap1 = _N, 4096
    idx1, bid1, v1, mvec1, offs1, jsel1 = _fps_level(
        pos, starts, counts, _RATIO1, cap1, ppad1, mcap1)
    nbr1, m1 = _radius_xla(pos, starts, counts, idx1, bid1, v1,
                           _r2_f32(_R1), _MAX_NB)
    pos1 = pos[idx1]
    starts1 = offs1[:-1]
    cap2 = cap1 // 4 + _B
    ppad2, mcap2 = 4224, 1024
    idx2, bid2, v2, mvec2, offs2, jsel2 = _fps_level(
        pos1, starts1, mvec1, _RATIO2, cap2, ppad2, mcap2)
    nbr2, m2 = _radius_xla(pos1, starts1, mvec1, idx2, bid2, v2,
                           _r2_f32(_R2), _MAX_NB)
    batch2 = jnp.where(v2, bid2, _B)

    rel1 = pos[nbr1] - pos1[:, None, :]
    x1 = _masked_max(_mlp_edge(rel1, params["local1"], m1), m1)
    pos2 = pos1[idx2]
    inp2 = jnp.concatenate([x1[nbr2], pos1[nbr2] - pos2[:, None, :]], axis=-1)
    x2 = _masked_max(_mlp_edge(inp2, params["local2"], m2), m2)
    h = jnp.concatenate([x2, pos2], axis=1)
    h = jnp.where(v2[:, None], h, 0.0)
    h = _mlp_node_masked(h, params["global"], v2)
    g = jax.ops.segment_max(h, batch2, num_segments=_B + 1)[:_B]
    out = _mlp_node(g, params["head"], False)
    return jax.nn.log_softmax(out, axis=-1)


# speedup vs baseline: 84.6482x; 70.2934x over previous
import functools

import jax
import jax.numpy as jnp
from jax import lax
from jax.experimental import pallas as pl
from jax.experimental.pallas import tpu as pltpu
from jax.experimental.pallas import tpu_sc as plsc

_RATIO1, _RATIO2 = 0.5, 0.25
_R1, _R2 = 0.2, 0.4
_MAX_NB = 64
_N, _B = 8192, 8

_NEG = float("-inf")


def _r2_f32(r):
    import numpy as np
    t = float(r) * float(r)
    t32 = np.float32(t)
    if float(t32) > t:
        t32 = np.nextafter(t32, np.float32(0.0))
    return t32


def _fps_body(mmax_ref, px_ref, py_ref, pz_ref, vm_ref, out_ref, dist_ref):
    B, P = px_ref.shape
    px = px_ref[...]
    py = py_ref[...]
    pz = pz_ref[...]
    d0 = ((px - px[:, 0:1]) ** 2 + (py - py[:, 0:1]) ** 2
          + (pz - pz[:, 0:1]) ** 2)
    dist_ref[...] = jnp.where(vm_ref[...] > 0, d0, _NEG)
    jidx = lax.broadcasted_iota(jnp.int32, (B, P), 1)
    lane = lax.broadcasted_iota(jnp.int32, (B, 128), 1)
    mmax = mmax_ref[0]

    def it(i, buf):
        dist = dist_ref[...]
        maxv = jnp.max(dist, axis=1, keepdims=True)
        sel = dist >= maxv
        jsel = jnp.min(jnp.where(sel, jidx, P), axis=1, keepdims=True)
        onehot = jidx == jsel
        qx = jnp.max(jnp.where(onehot, px, _NEG), axis=1, keepdims=True)
        qy = jnp.max(jnp.where(onehot, py, _NEG), axis=1, keepdims=True)
        qz = jnp.max(jnp.where(onehot, pz, _NEG), axis=1, keepdims=True)
        d2 = (px - qx) ** 2 + (py - qy) ** 2 + (pz - qz) ** 2
        dist_ref[...] = jnp.minimum(dist, d2)
        im = lax.rem(i, 128)
        buf = jnp.where(lane == im, jsel, buf)

        @pl.when(im == 127)
        def _flush():
            base = pl.multiple_of(i - 127, 128)
            out_ref[:, pl.ds(base, 128)] = buf

        return buf

    buf0 = jnp.zeros((B, 128), jnp.int32)
    buf = lax.fori_loop(1, mmax, it, buf0)
    last_base = pl.multiple_of(((mmax - 1) // 128) * 128, 128)
    out_ref[:, pl.ds(last_base, 128)] = buf


def _fps_pallas(px, py, pz, vmask, mmax, mcap, interpret=False):
    B, P = px.shape
    return pl.pallas_call(
        _fps_body,
        out_shape=jax.ShapeDtypeStruct((B, mcap), jnp.int32),
        in_specs=[
            pl.BlockSpec(memory_space=pltpu.SMEM),
            pl.BlockSpec(memory_space=pltpu.VMEM),
            pl.BlockSpec(memory_space=pltpu.VMEM),
            pl.BlockSpec(memory_space=pltpu.VMEM),
            pl.BlockSpec(memory_space=pltpu.VMEM),
        ],
        out_specs=pl.BlockSpec(memory_space=pltpu.VMEM),
        scratch_shapes=[pltpu.VMEM((B, P), jnp.float32)],
        interpret=interpret,
    )(jnp.reshape(mmax.astype(jnp.int32), (1,)), px, py, pz, vmask)


def _fps_level(pos_all, starts, counts, ratio, cap, ppad, mcap, interpret=False):
    P0 = pos_all.shape[0]
    mvec = jnp.maximum(1, jnp.ceil(ratio * counts.astype(jnp.float32)).astype(counts.dtype))
    offs = jnp.concatenate([jnp.zeros((1,), mvec.dtype), jnp.cumsum(mvec)])
    total = offs[-1]

    j = jnp.arange(ppad)
    gidx = jnp.minimum(starts[:, None] + j[None, :], P0 - 1)
    px = pos_all[:, 0][gidx]
    py = pos_all[:, 1][gidx]
    pz = pos_all[:, 2][gidx]
    vmask = (j[None, :] < counts[:, None]).astype(jnp.float32)
    mmax = jnp.max(mvec)

    jsel = _fps_pallas(px, py, pz, vmask, mmax, mcap, interpret=interpret)

    t = jnp.arange(cap)
    b = jnp.clip(jnp.searchsorted(offs, t, side="right") - 1, 0, _B - 1)
    i = jnp.clip(t - offs[b], 0, mcap - 1)
    jsel_t = jsel[b, i]
    gsel = jnp.minimum(starts[b] + jsel_t, P0 - 1)
    valid = t < total
    idx = jnp.where(valid, gsel, 0)
    bid = jnp.where(valid, b, 0).astype(starts.dtype)
    return idx, bid, valid, mvec, offs, jsel


_NBW = 80


def _radius_sc_call(qx, qy, qz, qs, qc, tx, ty, tz, thr, mpad, tlen):
    info = plsc.get_sparse_core_info()
    NW = info.num_cores * info.num_subcores
    L = info.num_lanes
    qpw = mpad // NW
    mesh = plsc.VectorSubcoreMesh(core_axis_name="c", subcore_axis_name="s")

    @functools.partial(
        pl.kernel, mesh=mesh,
        out_type=jax.ShapeDtypeStruct((mpad * _NBW,), jnp.int32),
        compiler_params=pltpu.CompilerParams(needs_layout_passes=False),
        scratch_types=[
            pltpu.VMEM((tlen,), jnp.float32),
            pltpu.VMEM((tlen,), jnp.float32),
            pltpu.VMEM((tlen,), jnp.float32),
            pltpu.VMEM((qpw,), jnp.float32),
            pltpu.VMEM((qpw,), jnp.float32),
            pltpu.VMEM((qpw,), jnp.float32),
            pltpu.VMEM((qpw,), jnp.int32),
            pltpu.VMEM((qpw,), jnp.int32),
            pltpu.VMEM((qpw * _NBW,), jnp.int32),
        ],
    )
    def k(qx_h, qy_h, qz_h, qs_h, qc_h, tx_h, ty_h, tz_h, out_h,
          txv, tyv, tzv, qxv, qyv, qzv, qsv, qcv, nbrv):
        wid = lax.axis_index("s") * info.num_cores + lax.axis_index("c")
        base = wid * qpw
        pltpu.sync_copy(tx_h, txv)
        pltpu.sync_copy(ty_h, tyv)
        pltpu.sync_copy(tz_h, tzv)
        pltpu.sync_copy(qx_h.at[pl.ds(base, qpw)], qxv)
        pltpu.sync_copy(qy_h.at[pl.ds(base, qpw)], qyv)
        pltpu.sync_copy(qz_h.at[pl.ds(base, qpw)], qzv)
        pltpu.sync_copy(qs_h.at[pl.ds(base, qpw)], qsv)
        pltpu.sync_copy(qc_h.at[pl.ds(base, qpw)], qcv)
        lanes = lax.broadcasted_iota(jnp.int32, (L,), 0)
        neg1 = jnp.full((L,), -1, jnp.int32)

        def per_group(g, carry):
            qx16 = qxv[pl.ds(g * L, L)]
            qy16 = qyv[pl.ds(g * L, L)]
            qz16 = qzv[pl.ds(g * L, L)]
            qs16 = qsv[pl.ds(g * L, L)]
            qc16 = qcv[pl.ds(g * L, L)]
            for lq in range(L):
                qxs = qx16[lq]
                qys = qy16[lq]
                qzs = qz16[lq]
                s0 = qs16[lq]
                cnt = qc16[lq]
                rowbase = (g * L + lq) * _NBW
                for s in range(4):
                    nbrv[pl.ds(rowbase + s * L, L)] = neg1
                c0 = s0 // L
                c1 = (s0 + cnt + L - 1) // L

                def body(c, off, s0=s0, cnt=cnt, qxs=qxs, qys=qys, qzs=qzs,
                         rowbase=rowbase):
                    absj = lanes + c * L
                    x = txv[pl.ds(c * L, L)]
                    y = tyv[pl.ds(c * L, L)]
                    z = tzv[pl.ds(c * L, L)]
                    dx = x - qxs
                    dy = y - qys
                    dz = z - qzs
                    d2 = dx * dx + dy * dy + dz * dz
                    mi = ((absj >= s0) & (absj < s0 + cnt)
                          & (d2 <= thr)).astype(jnp.int32)
                    rank = lax.cumsum(mi) - mi
                    mst = (mi > 0) & (off + rank < _MAX_NB)
                    offc = jnp.minimum(off, _MAX_NB)
                    plsc.store_compressed(nbrv.at[pl.ds(rowbase + offc, L)],
                                          absj, mask=mst)
                    return off + jnp.sum(mi)

                lax.fori_loop(c0, c1, body, jnp.int32(0))
            return carry

        lax.fori_loop(0, qpw // L, per_group, 0)
        pltpu.sync_copy(nbrv, out_h.at[pl.ds(base * _NBW, qpw * _NBW)])

    flat = k(qx, qy, qz, qs, qc, tx, ty, tz)
    return flat.reshape(mpad, _NBW)


def _radius_sc(pos_all, starts, counts, q_idx, q_bid, q_val, thr, max_nb,
               mpad, tlen):
    M = q_idx.shape[0]
    P0 = pos_all.shape[0]
    qpos = pos_all[q_idx]
    qs = starts[q_bid].astype(jnp.int32)
    qc = jnp.where(q_val, counts[q_bid], 0).astype(jnp.int32)
    pad = mpad - M
    qx = jnp.pad(qpos[:, 0], (0, pad))
    qy = jnp.pad(qpos[:, 1], (0, pad))
    qz = jnp.pad(qpos[:, 2], (0, pad))
    qs = jnp.pad(qs, (0, pad))
    qc = jnp.pad(qc, (0, pad))
    tpad = tlen - P0
    tx = jnp.pad(pos_all[:, 0], (0, tpad))
    ty = jnp.pad(pos_all[:, 1], (0, tpad))
    tz = jnp.pad(pos_all[:, 2], (0, tpad))
    nbr_raw = _radius_sc_call(qx, qy, qz, qs, qc, tx, ty, tz,
                              float(thr), mpad, tlen)[:M, :max_nb]
    msk = nbr_raw >= 0
    nbr = jnp.where(msk, nbr_raw, 0)
    return nbr, msk


def _radius_xla(pos_all, starts, counts, q_idx, q_bid, q_val, thr, max_nb):
    P = pos_all.shape[0]
    j = jnp.arange(P)

    def one(qi, qb, qv):
        g = jnp.minimum(starts[qb] + j, P - 1)
        d2 = jnp.sum((pos_all[g] - pos_all[qi]) ** 2, axis=-1)
        cond = (j < counts[qb]) & (d2 <= thr) & qv
        rank = jnp.cumsum(cond.astype(jnp.int32)) - 1
        slot = jnp.where(cond & (rank < max_nb), rank, max_nb)
        nbr = jnp.zeros((max_nb + 1,), g.dtype).at[slot].set(g)[:max_nb]
        cnt = jnp.minimum(jnp.sum(cond.astype(jnp.int32)), max_nb)
        msk = jnp.arange(max_nb) < cnt
        return nbr, msk

    return jax.vmap(one)(q_idx, q_bid, q_val)


def _masked_bn(h, mask, g, be, eps=1e-5):
    m = mask[..., None].astype(h.dtype)
    cnt = jnp.maximum(jnp.sum(m), 1.0)
    mean = jnp.sum(h * m, axis=(0, 1)) / cnt
    var = jnp.sum(((h - mean) ** 2) * m, axis=(0, 1)) / cnt
    return (h - mean) / jnp.sqrt(var + eps) * g + be


def _mlp_edge(h, layers, mask):
    for i, lp in enumerate(layers):
        h = h @ lp["W"] + lp["b"]
        if i < len(layers) - 1:
            h = _masked_bn(h, mask, lp["g"], lp["be"])
            h = jax.nn.relu(h)
    return h


def _mlp_node(h, layers, use_bn):
    for i, lp in enumerate(layers):
        h = h @ lp["W"] + lp["b"]
        if i < len(layers) - 1:
            if use_bn:
                mean = jnp.mean(h, axis=0)
                var = jnp.mean((h - mean) ** 2, axis=0)
                h = (h - mean) / jnp.sqrt(var + 1e-5) * lp["g"] + lp["be"]
            h = jax.nn.relu(h)
    return h


def _mlp_node_masked(h, layers, valid, eps=1e-5):
    m = valid[:, None].astype(h.dtype)
    cnt = jnp.maximum(jnp.sum(m), 1.0)
    for i, lp in enumerate(layers):
        h = h @ lp["W"] + lp["b"]
        if i < len(layers) - 1:
            mean = jnp.sum(h * m, axis=0) / cnt
            var = jnp.sum(((h - mean) ** 2) * m, axis=0) / cnt
            h = (h - mean) / jnp.sqrt(var + eps) * lp["g"] + lp["be"]
            h = jax.nn.relu(h)
    return h


def _masked_max(h, mask):
    return jnp.max(jnp.where(mask[..., None], h, jnp.finfo(h.dtype).min), axis=1)


def kernel(pos, batch, params):
    n = pos.shape[0]
    counts = jnp.bincount(batch, length=_B)
    starts = jnp.concatenate([jnp.zeros((1,), counts.dtype), jnp.cumsum(counts)[:-1]])
    cap1 = n // 2 + _B
    ppad1, mcap1 = _N, 4096
    idx1, bid1, v1, mvec1, offs1, jsel1 = _fps_level(
        pos, starts, counts, _RATIO1, cap1, ppad1, mcap1)
    nbr1, m1 = _radius_sc(pos, starts, counts, idx1, bid1, v1,
                          _r2_f32(_R1), _MAX_NB, 4608, _N)
    pos1 = pos[idx1]
    starts1 = offs1[:-1]
    cap2 = cap1 // 4 + _B
    ppad2, mcap2 = 4224, 1024
    idx2, bid2, v2, mvec2, offs2, jsel2 = _fps_level(
        pos1, starts1, mvec1, _RATIO2, cap2, ppad2, mcap2)
    nbr2, m2 = _radius_sc(pos1, starts1, mvec1, idx2, bid2, v2,
                          _r2_f32(_R2), _MAX_NB, 1536, 4224)
    batch2 = jnp.where(v2, bid2, _B)

    rel1 = pos[nbr1] - pos1[:, None, :]
    x1 = _masked_max(_mlp_edge(rel1, params["local1"], m1), m1)
    pos2 = pos1[idx2]
    inp2 = jnp.concatenate([x1[nbr2], pos1[nbr2] - pos2[:, None, :]], axis=-1)
    x2 = _masked_max(_mlp_edge(inp2, params["local2"], m2), m2)
    h = jnp.concatenate([x2, pos2], axis=1)
    h = jnp.where(v2[:, None], h, 0.0)
    h = _mlp_node_masked(h, params["global"], v2)
    g = jax.ops.segment_max(h, batch2, num_segments=_B + 1)[:_B]
    out = _mlp_node(g, params["head"], False)
    return jax.nn.log_softmax(out, axis=-1)
